# baseline (speedup 1.0000x reference)
import jax
import jax.numpy as jnp
from jax import lax
from jax.experimental import pallas as pl
from jax.experimental.pallas import tpu as pltpu
from jax.experimental.pallas import tpu_sc as plsc

N = 10000
E = 320000
D = 128
G = 64
CLS = 64

NC = 2
NS = 16
NW = NC * NS
N_PAD = 10240
EW = E // NW
CHUNK = 125
BLK = 5
NCHUNK = EW // CHUNK
NGRP = NCHUNK // BLK
RPS = N_PAD // NS

R = 1024
NB = N_PAD // R


def _agg_body(with_deg, h_hbm, src_hbm, dst_hbm, out_hbm, deg_hbm,
              sblk, dblk, rows, sem_g, sem_s, sem_i, acc, ones_v, deg_sh):
    cid = lax.axis_index("c")
    sid = lax.axis_index("s")
    wid = sid * NC + cid

    zeros16 = jnp.zeros((16,), jnp.float32)

    def _zrow(r, carry):
        for c8 in range(D // 16):
            rows[0][r, pl.ds(c8 * 16, 16)] = zeros16
        return carry
    lax.fori_loop(0, CHUNK, _zrow, 0)
    base = sid * RPS
    for k in range(RPS // CHUNK):
        pltpu.sync_copy(rows[0], acc.at[pl.ds(base + k * CHUNK, CHUNK)])
    rem = RPS - (RPS // CHUNK) * CHUNK
    if rem:
        pltpu.sync_copy(rows[0].at[pl.ds(0, rem)],
                        acc.at[pl.ds(base + RPS - rem, rem)])

    if with_deg:
        ones16 = jnp.ones((16,), jnp.float32)
        for k in range(8):
            ones_v[pl.ds(k * 16, 16)] = ones16
        for k in range(RPS // D):
            pltpu.sync_copy(rows[0].at[0],
                            deg_sh.at[pl.ds(base + k * D, D)])

    pltpu.sync_copy(src_hbm.at[wid, 0], sblk[0])
    pltpu.sync_copy(dst_hbm.at[wid, 0], dblk[0])
    pltpu.async_copy(src_hbm.at[wid, 1], sblk[1], sem_i[1])
    pltpu.async_copy(dst_hbm.at[wid, 1], dblk[1], sem_i[1])

    pltpu.async_copy(h_hbm.at[sblk[0].at[0]], rows[0], sem_g[0])

    plsc.subcore_barrier()

    def _outer(tt, carry):
        for p in range(2):
            t = tt * 2 + p

            @pl.when(t + 1 < NGRP)
            def _():
                nt = t + 1
                pltpu.make_async_copy(src_hbm.at[wid, nt],
                                      sblk[1 - p], sem_i[1 - p]).wait()
                pltpu.make_async_copy(dst_hbm.at[wid, nt],
                                      dblk[1 - p], sem_i[1 - p]).wait()

            for b in range(BLK):
                j = t * BLK + b
                rb = (p * BLK + b) % 2
                nrb = 1 - rb
                gq, gr = (p, b + 1) if b + 1 < BLK else (1 - p, 0)

                pltpu.make_async_copy(h_hbm.at[sblk[p].at[b]], rows[rb],
                                      sem_g[rb]).wait()
                pltpu.async_copy(rows[rb], acc.at[dblk[p].at[b]], sem_s[rb],
                                 add=True)
                if with_deg:
                    pltpu.sync_copy(ones_v.at[pl.ds(0, CHUNK)],
                                    deg_sh.at[dblk[p].at[b]], add=True)

                @pl.when(j >= 1)
                def _():
                    pltpu.make_async_copy(rows[nrb], acc.at[dblk[p].at[b]],
                                          sem_s[nrb]).wait()

                @pl.when(j + 1 < NCHUNK)
                def _():
                    pltpu.async_copy(h_hbm.at[sblk[gq].at[gr]], rows[nrb],
                                     sem_g[nrb])

            @pl.when(t + 2 < NGRP)
            def _():
                nt2 = t + 2
                pltpu.async_copy(src_hbm.at[wid, nt2], sblk[p], sem_i[p])
                pltpu.async_copy(dst_hbm.at[wid, nt2], dblk[p], sem_i[p])
        return carry
    lax.fori_loop(0, NGRP // 2, _outer, 0)

    lp = (NCHUNK - 1) % 2
    pltpu.make_async_copy(rows[lp], acc.at[dblk[0].at[0]], sem_s[lp]).wait()

    plsc.subcore_barrier()

    pltpu.sync_copy(acc.at[pl.ds(base, RPS)],
                    out_hbm.at[cid, pl.ds(base, RPS)])
    if with_deg:
        pltpu.sync_copy(deg_sh.at[pl.ds(base, RPS)],
                        deg_hbm.at[cid, pl.ds(base, RPS)])


def _make_agg(with_deg):
    mesh = plsc.VectorSubcoreMesh(core_axis_name="c", subcore_axis_name="s")
    out_type = [jax.ShapeDtypeStruct((NC, N_PAD, D), jnp.float32)]
    scratch = [pltpu.VMEM((BLK, CHUNK), jnp.int32) for _ in range(4)]
    scratch += [pltpu.VMEM((CHUNK, D), jnp.float32) for _ in range(2)]
    scratch += [pltpu.SemaphoreType.DMA for _ in range(6)]
    scratch.append(pltpu.VMEM_SHARED((N_PAD, D), jnp.float32))
    if with_deg:
        out_type.append(jax.ShapeDtypeStruct((NC, N_PAD), jnp.float32))
        scratch.append(pltpu.VMEM((128,), jnp.float32))
        scratch.append(pltpu.VMEM_SHARED((N_PAD,), jnp.float32))

        def body(h, src, dst, out, deg, *rest):
            _split(True, h, src, dst, out, deg, rest)
    else:
        def body(h, src, dst, out, *rest):
            _split(False, h, src, dst, out, None, rest)

    def _split(wd, h, src, dst, out, deg, rest):
        sblk = rest[0:2]
        dblk = rest[2:4]
        rows = rest[4:6]
        sem_g = rest[6:8]
        sem_s = rest[8:10]
        sem_i = rest[10:12]
        acc = rest[12]
        ones_v = rest[13] if wd else None
        deg_sh = rest[14] if wd else None
        _agg_body(wd, h, src, dst, out, deg, sblk, dblk, rows, sem_g, sem_s,
                  sem_i, acc, ones_v, deg_sh)

    out_type = tuple(out_type) if with_deg else out_type[0]
    return pl.kernel(body, out_type=out_type, mesh=mesh,
                     scratch_types=tuple(scratch))


_agg_deg = _make_agg(True)
_agg = _make_agg(False)


def _layer_body(p_ref, degp_ref, x_ref, wl_ref, wr_ref, b_ref, o_ref):
    deg = jnp.sum(degp_ref[...], axis=0)
    inv = 1.0 / jnp.maximum(deg, 1.0)
    mean = (p_ref[0] + p_ref[1]) * inv[:, None]
    h = jnp.dot(mean, wl_ref[...]) + jnp.dot(x_ref[...], wr_ref[...]) + b_ref[...]
    o_ref[...] = jnp.maximum(h, 0.0)


def _layer(partials, deg_parts, x, wl, wr, b):
    return pl.pallas_call(
        _layer_body,
        grid=(NB,),
        in_specs=[
            pl.BlockSpec((NC, R, D), lambda i: (0, i, 0)),
            pl.BlockSpec((NC, R), lambda i: (0, i)),
            pl.BlockSpec((R, D), lambda i: (i, 0)),
            pl.BlockSpec((D, D), lambda i: (0, 0)),
            pl.BlockSpec((D, D), lambda i: (0, 0)),
            pl.BlockSpec((1, D), lambda i: (0, 0)),
        ],
        out_specs=pl.BlockSpec((R, D), lambda i: (i, 0)),
        out_shape=jax.ShapeDtypeStruct((N_PAD, D), jnp.float32),
    )(partials, deg_parts, x, wl, wr, b)


def _final_body(p_ref, degp_ref, h_ref, wl_ref, wr_ref, b_ref, batch_ref,
                w1_ref, b1_ref, w2_ref, b2_ref, o_ref, pool_scr, cnt_scr):
    i = pl.program_id(0)
    deg = jnp.sum(degp_ref[...], axis=0)
    inv = 1.0 / jnp.maximum(deg, 1.0)
    mean = (p_ref[0] + p_ref[1]) * inv[:, None]
    h3 = jnp.maximum(
        jnp.dot(mean, wl_ref[...]) + jnp.dot(h_ref[...], wr_ref[...])
        + b_ref[...], 0.0)
    bvec = batch_ref[0, 0, :]
    oh = (bvec[None, :] == lax.broadcasted_iota(jnp.int32, (G, R), 0)
          ).astype(jnp.float32)
    pool_upd = jnp.dot(oh, h3)
    cnt_upd = jnp.sum(oh, axis=1)[None, :]

    @pl.when(i == 0)
    def _():
        pool_scr[...] = pool_upd
        cnt_scr[...] = cnt_upd

    @pl.when(i > 0)
    def _():
        pool_scr[...] += pool_upd
        cnt_scr[...] += cnt_upd

    @pl.when(i == NB - 1)
    def _():
        pooled = pool_scr[...] / jnp.maximum(cnt_scr[0, :], 1.0)[:, None]
        z = jnp.maximum(jnp.dot(pooled, w1_ref[...]) + b1_ref[...], 0.0)
        logits = jnp.dot(z, w2_ref[...]) + b2_ref[...]
        m = jnp.max(logits, axis=-1, keepdims=True)
        s = jnp.log(jnp.sum(jnp.exp(logits - m), axis=-1, keepdims=True))
        o_ref[...] = logits - m - s


def _final(partials, deg_parts, h, wl, wr, b, batch3d, w1, b1, w2, b2):
    return pl.pallas_call(
        _final_body,
        grid=(NB,),
        in_specs=[
            pl.BlockSpec((NC, R, D), lambda i: (0, i, 0)),
            pl.BlockSpec((NC, R), lambda i: (0, i)),
            pl.BlockSpec((R, D), lambda i: (i, 0)),
            pl.BlockSpec((D, D), lambda i: (0, 0)),
            pl.BlockSpec((D, D), lambda i: (0, 0)),
            pl.BlockSpec((1, D), lambda i: (0, 0)),
            pl.BlockSpec((1, 1, R), lambda i: (i, 0, 0)),
            pl.BlockSpec((D, D), lambda i: (0, 0)),
            pl.BlockSpec((1, D), lambda i: (0, 0)),
            pl.BlockSpec((D, CLS), lambda i: (0, 0)),
            pl.BlockSpec((1, CLS), lambda i: (0, 0)),
        ],
        out_specs=pl.BlockSpec((G, CLS), lambda i: (0, 0)),
        out_shape=jax.ShapeDtypeStruct((G, CLS), jnp.float32),
        scratch_shapes=[
            pltpu.VMEM((G, D), jnp.float32),
            pltpu.VMEM((1, G), jnp.float32),
        ],
    )(partials, deg_parts, h, wl, wr, b, batch3d, w1, b1, w2, b2)


def kernel(x, edge_index, batch, Wl1, Wr1, b1, Wl2, Wr2, b2, Wl3, Wr3, b3,
           W_lin1, b_lin1, W_lin2, b_lin2):
    srcr = edge_index[0].reshape(NW, NGRP, BLK, CHUNK)
    dstr = edge_index[1].reshape(NW, NGRP, BLK, CHUNK)
    x_pad = jnp.pad(x, ((0, N_PAD - N), (0, 0)))
    batch3d = jnp.pad(batch, (0, N_PAD - N), constant_values=G
                      ).reshape(NB, 1, R)

    agg1, degp = _agg_deg(x_pad, srcr, dstr)
    h1 = _layer(agg1, degp, x_pad, Wl1, Wr1, b1.reshape(1, D))
    agg2 = _agg(h1, srcr, dstr)
    h2 = _layer(agg2, degp, h1, Wl2, Wr2, b2.reshape(1, D))
    agg3 = _agg(h2, srcr, dstr)
    return _final(agg3, degp, h2, Wl3, Wr3, b3.reshape(1, D), batch3d,
                  W_lin1, b_lin1.reshape(1, D), W_lin2, b_lin2.reshape(1, CLS))


# speedup vs baseline: 1.1522x; 1.1522x over previous
import jax
import jax.numpy as jnp
from jax import lax
from jax.experimental import pallas as pl
from jax.experimental.pallas import tpu as pltpu
from jax.experimental.pallas import tpu_sc as plsc

N = 10000
E = 320000
D = 128
G = 64
CLS = 64

NC = 2
NS = 16
NW = NC * NS
N_PAD = 10240
EW = E // NW
CHUNK = 125
BLK = 5
NCHUNK = EW // CHUNK
NGRP = NCHUNK // BLK
RPS = N_PAD // NS

R = 1024
NB = N_PAD // R


def _agg_body(with_deg, h_hbm, src_hbm, dst_hbm, out_hbm, deg_hbm,
              sblk, dblk, rows, sem_g, sem_s, sem_i, acc, ones_v, deg_sh):
    cid = lax.axis_index("c")
    sid = lax.axis_index("s")
    wid = sid * NC + cid

    zeros16 = jnp.zeros((16,), jnp.float32)

    def _zrow(r, carry):
        for c8 in range(D // 16):
            rows[0][r, pl.ds(c8 * 16, 16)] = zeros16
        return carry
    lax.fori_loop(0, CHUNK, _zrow, 0)
    base = sid * RPS
    for k in range(RPS // CHUNK):
        pltpu.sync_copy(rows[0], acc.at[pl.ds(base + k * CHUNK, CHUNK)])
    rem = RPS - (RPS // CHUNK) * CHUNK
    if rem:
        pltpu.sync_copy(rows[0].at[pl.ds(0, rem)],
                        acc.at[pl.ds(base + RPS - rem, rem)])

    if with_deg:
        ones16 = jnp.ones((16,), jnp.float32)
        for k in range(8):
            ones_v[pl.ds(k * 16, 16)] = ones16
        for k in range(RPS // D):
            pltpu.sync_copy(rows[0].at[0],
                            deg_sh.at[pl.ds(base + k * D, D)])

    pltpu.sync_copy(src_hbm.at[wid, 0], sblk[0])
    pltpu.sync_copy(dst_hbm.at[wid, 0], dblk[0])
    pltpu.async_copy(src_hbm.at[wid, 1], sblk[1], sem_i[1])
    pltpu.async_copy(dst_hbm.at[wid, 1], dblk[1], sem_i[1])

    pltpu.async_copy(h_hbm.at[sblk[0].at[0]], rows[0], sem_g[0])

    plsc.subcore_barrier()

    def _outer(tt, carry):
        for p in range(2):
            t = tt * 2 + p

            @pl.when(t + 1 < NGRP)
            def _():
                nt = t + 1
                pltpu.make_async_copy(src_hbm.at[wid, nt],
                                      sblk[1 - p], sem_i[1 - p]).wait()
                pltpu.make_async_copy(dst_hbm.at[wid, nt],
                                      dblk[1 - p], sem_i[1 - p]).wait()

            for b in range(BLK):
                j = t * BLK + b
                rb = (p * BLK + b) % 2
                nrb = 1 - rb
                gq, gr = (p, b + 1) if b + 1 < BLK else (1 - p, 0)

                @pl.when(j + 1 < NCHUNK)
                def _():
                    pltpu.async_copy(h_hbm.at[sblk[gq].at[gr]], rows[nrb],
                                     sem_g[nrb])

                pltpu.make_async_copy(h_hbm.at[sblk[p].at[b]], rows[rb],
                                      sem_g[rb]).wait()
                pltpu.sync_copy(rows[rb], acc.at[dblk[p].at[b]], add=True)
                if with_deg:
                    pltpu.sync_copy(ones_v.at[pl.ds(0, CHUNK)],
                                    deg_sh.at[dblk[p].at[b]], add=True)

            @pl.when(t + 2 < NGRP)
            def _():
                nt2 = t + 2
                pltpu.async_copy(src_hbm.at[wid, nt2], sblk[p], sem_i[p])
                pltpu.async_copy(dst_hbm.at[wid, nt2], dblk[p], sem_i[p])
        return carry
    lax.fori_loop(0, NGRP // 2, _outer, 0)

    plsc.subcore_barrier()

    pltpu.sync_copy(acc.at[pl.ds(base, RPS)],
                    out_hbm.at[cid, pl.ds(base, RPS)])
    if with_deg:
        pltpu.sync_copy(deg_sh.at[pl.ds(base, RPS)],
                        deg_hbm.at[cid, pl.ds(base, RPS)])


def _make_agg(with_deg):
    mesh = plsc.VectorSubcoreMesh(core_axis_name="c", subcore_axis_name="s")
    out_type = [jax.ShapeDtypeStruct((NC, N_PAD, D), jnp.float32)]
    scratch = [pltpu.VMEM((BLK, CHUNK), jnp.int32) for _ in range(4)]
    scratch += [pltpu.VMEM((CHUNK, D), jnp.float32) for _ in range(2)]
    scratch += [pltpu.SemaphoreType.DMA for _ in range(6)]
    scratch.append(pltpu.VMEM_SHARED((N_PAD, D), jnp.float32))
    if with_deg:
        out_type.append(jax.ShapeDtypeStruct((NC, N_PAD), jnp.float32))
        scratch.append(pltpu.VMEM((128,), jnp.float32))
        scratch.append(pltpu.VMEM_SHARED((N_PAD,), jnp.float32))

        def body(h, src, dst, out, deg, *rest):
            _split(True, h, src, dst, out, deg, rest)
    else:
        def body(h, src, dst, out, *rest):
            _split(False, h, src, dst, out, None, rest)

    def _split(wd, h, src, dst, out, deg, rest):
        sblk = rest[0:2]
        dblk = rest[2:4]
        rows = rest[4:6]
        sem_g = rest[6:8]
        sem_s = rest[8:10]
        sem_i = rest[10:12]
        acc = rest[12]
        ones_v = rest[13] if wd else None
        deg_sh = rest[14] if wd else None
        _agg_body(wd, h, src, dst, out, deg, sblk, dblk, rows, sem_g, sem_s,
                  sem_i, acc, ones_v, deg_sh)

    out_type = tuple(out_type) if with_deg else out_type[0]
    return pl.kernel(body, out_type=out_type, mesh=mesh,
                     scratch_types=tuple(scratch))


_agg_deg = _make_agg(True)
_agg = _make_agg(False)


def _layer_body(p_ref, degp_ref, x_ref, wl_ref, wr_ref, b_ref, o_ref):
    deg = jnp.sum(degp_ref[...], axis=0)
    inv = 1.0 / jnp.maximum(deg, 1.0)
    mean = (p_ref[0] + p_ref[1]) * inv[:, None]
    h = jnp.dot(mean, wl_ref[...]) + jnp.dot(x_ref[...], wr_ref[...]) + b_ref[...]
    o_ref[...] = jnp.maximum(h, 0.0)


def _layer(partials, deg_parts, x, wl, wr, b):
    return pl.pallas_call(
        _layer_body,
        grid=(NB,),
        in_specs=[
            pl.BlockSpec((NC, R, D), lambda i: (0, i, 0)),
            pl.BlockSpec((NC, R), lambda i: (0, i)),
            pl.BlockSpec((R, D), lambda i: (i, 0)),
            pl.BlockSpec((D, D), lambda i: (0, 0)),
            pl.BlockSpec((D, D), lambda i: (0, 0)),
            pl.BlockSpec((1, D), lambda i: (0, 0)),
        ],
        out_specs=pl.BlockSpec((R, D), lambda i: (i, 0)),
        out_shape=jax.ShapeDtypeStruct((N_PAD, D), jnp.float32),
    )(partials, deg_parts, x, wl, wr, b)


def _final_body(p_ref, degp_ref, h_ref, wl_ref, wr_ref, b_ref, batch_ref,
                w1_ref, b1_ref, w2_ref, b2_ref, o_ref, pool_scr, cnt_scr):
    i = pl.program_id(0)
    deg = jnp.sum(degp_ref[...], axis=0)
    inv = 1.0 / jnp.maximum(deg, 1.0)
    mean = (p_ref[0] + p_ref[1]) * inv[:, None]
    h3 = jnp.maximum(
        jnp.dot(mean, wl_ref[...]) + jnp.dot(h_ref[...], wr_ref[...])
        + b_ref[...], 0.0)
    bvec = batch_ref[0, 0, :]
    oh = (bvec[None, :] == lax.broadcasted_iota(jnp.int32, (G, R), 0)
          ).astype(jnp.float32)
    pool_upd = jnp.dot(oh, h3)
    cnt_upd = jnp.sum(oh, axis=1)[None, :]

    @pl.when(i == 0)
    def _():
        pool_scr[...] = pool_upd
        cnt_scr[...] = cnt_upd

    @pl.when(i > 0)
    def _():
        pool_scr[...] += pool_upd
        cnt_scr[...] += cnt_upd

    @pl.when(i == NB - 1)
    def _():
        pooled = pool_scr[...] / jnp.maximum(cnt_scr[0, :], 1.0)[:, None]
        z = jnp.maximum(jnp.dot(pooled, w1_ref[...]) + b1_ref[...], 0.0)
        logits = jnp.dot(z, w2_ref[...]) + b2_ref[...]
        m = jnp.max(logits, axis=-1, keepdims=True)
        s = jnp.log(jnp.sum(jnp.exp(logits - m), axis=-1, keepdims=True))
        o_ref[...] = logits - m - s


def _final(partials, deg_parts, h, wl, wr, b, batch3d, w1, b1, w2, b2):
    return pl.pallas_call(
        _final_body,
        grid=(NB,),
        in_specs=[
            pl.BlockSpec((NC, R, D), lambda i: (0, i, 0)),
            pl.BlockSpec((NC, R), lambda i: (0, i)),
            pl.BlockSpec((R, D), lambda i: (i, 0)),
            pl.BlockSpec((D, D), lambda i: (0, 0)),
            pl.BlockSpec((D, D), lambda i: (0, 0)),
            pl.BlockSpec((1, D), lambda i: (0, 0)),
            pl.BlockSpec((1, 1, R), lambda i: (i, 0, 0)),
            pl.BlockSpec((D, D), lambda i: (0, 0)),
            pl.BlockSpec((1, D), lambda i: (0, 0)),
            pl.BlockSpec((D, CLS), lambda i: (0, 0)),
            pl.BlockSpec((1, CLS), lambda i: (0, 0)),
        ],
        out_specs=pl.BlockSpec((G, CLS), lambda i: (0, 0)),
        out_shape=jax.ShapeDtypeStruct((G, CLS), jnp.float32),
        scratch_shapes=[
            pltpu.VMEM((G, D), jnp.float32),
            pltpu.VMEM((1, G), jnp.float32),
        ],
    )(partials, deg_parts, h, wl, wr, b, batch3d, w1, b1, w2, b2)


def kernel(x, edge_index, batch, Wl1, Wr1, b1, Wl2, Wr2, b2, Wl3, Wr3, b3,
           W_lin1, b_lin1, W_lin2, b_lin2):
    srcr = edge_index[0].reshape(NW, NGRP, BLK, CHUNK)
    dstr = edge_index[1].reshape(NW, NGRP, BLK, CHUNK)
    x_pad = jnp.pad(x, ((0, N_PAD - N), (0, 0)))
    batch3d = jnp.pad(batch, (0, N_PAD - N), constant_values=G
                      ).reshape(NB, 1, R)

    agg1, degp = _agg_deg(x_pad, srcr, dstr)
    h1 = _layer(agg1, degp, x_pad, Wl1, Wr1, b1.reshape(1, D))
    agg2 = _agg(h1, srcr, dstr)
    h2 = _layer(agg2, degp, h1, Wl2, Wr2, b2.reshape(1, D))
    agg3 = _agg(h2, srcr, dstr)
    return _final(agg3, degp, h2, Wl3, Wr3, b3.reshape(1, D), batch3d,
                  W_lin1, b_lin1.reshape(1, D), W_lin2, b_lin2.reshape(1, CLS))


# speedup vs baseline: 1.1537x; 1.0013x over previous
import jax
import jax.numpy as jnp
from jax import lax
from jax.experimental import pallas as pl
from jax.experimental.pallas import tpu as pltpu
from jax.experimental.pallas import tpu_sc as plsc

N = 10000
E = 320000
D = 128
G = 64
CLS = 64

NC = 2
NS = 16
NW = NC * NS
N_PAD = 10240
EW = E // NW
CHUNK = 125
BLK = 5
NCHUNK = EW // CHUNK
NGRP = NCHUNK // BLK
RPS = N_PAD // NS

R = 1024
NB = N_PAD // R


def _agg_body(with_deg, h_hbm, src_hbm, dst_hbm, out_hbm, deg_hbm,
              sblk, dblk, rows, sem_g, sem_i, acc, ones_v, deg_sh):
    cid = lax.axis_index("c")
    sid = lax.axis_index("s")
    wid = sid * NC + cid

    zeros16 = jnp.zeros((16,), jnp.float32)

    def _zrow(r, carry):
        for c8 in range(D // 16):
            rows[0][r, pl.ds(c8 * 16, 16)] = zeros16
        return carry
    lax.fori_loop(0, CHUNK, _zrow, 0)
    base = sid * RPS
    for k in range(RPS // CHUNK):
        pltpu.sync_copy(rows[0], acc.at[pl.ds(base + k * CHUNK, CHUNK)])
    rem = RPS - (RPS // CHUNK) * CHUNK
    if rem:
        pltpu.sync_copy(rows[0].at[pl.ds(0, rem)],
                        acc.at[pl.ds(base + RPS - rem, rem)])

    if with_deg:
        ones16 = jnp.ones((16,), jnp.float32)
        for k in range(8):
            ones_v[pl.ds(k * 16, 16)] = ones16
        for k in range(RPS // D):
            pltpu.sync_copy(rows[0].at[0],
                            deg_sh.at[pl.ds(base + k * D, D)])

    pltpu.sync_copy(src_hbm.at[wid, 0], sblk[0])
    pltpu.sync_copy(dst_hbm.at[wid, 0], dblk[0])
    pltpu.async_copy(src_hbm.at[wid, 1], sblk[1], sem_i[1])
    pltpu.async_copy(dst_hbm.at[wid, 1], dblk[1], sem_i[1])

    pltpu.async_copy(h_hbm.at[sblk[0].at[0]], rows[0], sem_g[0])

    plsc.subcore_barrier()

    def _outer(tt, carry):
        for p in range(2):
            t = tt * 2 + p

            @pl.when(t + 1 < NGRP)
            def _():
                nt = t + 1
                pltpu.make_async_copy(src_hbm.at[wid, nt],
                                      sblk[1 - p], sem_i[1 - p]).wait()
                pltpu.make_async_copy(dst_hbm.at[wid, nt],
                                      dblk[1 - p], sem_i[1 - p]).wait()

            for b in range(BLK):
                j = t * BLK + b
                rb = (p * BLK + b) % 2
                nrb = 1 - rb
                gq, gr = (p, b + 1) if b + 1 < BLK else (1 - p, 0)

                @pl.when(j + 1 < NCHUNK)
                def _():
                    pltpu.async_copy(h_hbm.at[sblk[gq].at[gr]], rows[nrb],
                                     sem_g[nrb])

                pltpu.make_async_copy(h_hbm.at[sblk[p].at[b]], rows[rb],
                                      sem_g[rb]).wait()
                pltpu.sync_copy(rows[rb], acc.at[dblk[p].at[b]], add=True)
                if with_deg:
                    pltpu.sync_copy(ones_v.at[pl.ds(0, CHUNK)],
                                    deg_sh.at[dblk[p].at[b]], add=True)

            @pl.when(t + 2 < NGRP)
            def _():
                nt2 = t + 2
                pltpu.async_copy(src_hbm.at[wid, nt2], sblk[p], sem_i[p])
                pltpu.async_copy(dst_hbm.at[wid, nt2], dblk[p], sem_i[p])
        return carry
    lax.fori_loop(0, NGRP // 2, _outer, 0)

    plsc.subcore_barrier()

    pltpu.sync_copy(acc.at[pl.ds(base, RPS)],
                    out_hbm.at[cid, pl.ds(base, RPS)])
    if with_deg:
        pltpu.sync_copy(deg_sh.at[pl.ds(base, RPS)],
                        deg_hbm.at[cid, pl.ds(base, RPS)])


def _make_agg(with_deg):
    mesh = plsc.VectorSubcoreMesh(core_axis_name="c", subcore_axis_name="s")
    out_type = [jax.ShapeDtypeStruct((NC, N_PAD, D), jnp.float32)]
    scratch = [pltpu.VMEM((BLK, CHUNK), jnp.int32) for _ in range(4)]
    scratch += [pltpu.VMEM((CHUNK, D), jnp.float32) for _ in range(2)]
    scratch += [pltpu.SemaphoreType.DMA for _ in range(4)]
    scratch.append(pltpu.VMEM_SHARED((N_PAD, D), jnp.float32))
    if with_deg:
        out_type.append(jax.ShapeDtypeStruct((NC, N_PAD), jnp.float32))
        scratch.append(pltpu.VMEM((128,), jnp.float32))
        scratch.append(pltpu.VMEM_SHARED((N_PAD,), jnp.float32))

        def body(h, src, dst, out, deg, *rest):
            _split(True, h, src, dst, out, deg, rest)
    else:
        def body(h, src, dst, out, *rest):
            _split(False, h, src, dst, out, None, rest)

    def _split(wd, h, src, dst, out, deg, rest):
        sblk = rest[0:2]
        dblk = rest[2:4]
        rows = rest[4:6]
        sem_g = rest[6:8]
        sem_i = rest[8:10]
        acc = rest[10]
        ones_v = rest[11] if wd else None
        deg_sh = rest[12] if wd else None
        _agg_body(wd, h, src, dst, out, deg, sblk, dblk, rows, sem_g,
                  sem_i, acc, ones_v, deg_sh)

    out_type = tuple(out_type) if with_deg else out_type[0]
    return pl.kernel(body, out_type=out_type, mesh=mesh,
                     scratch_types=tuple(scratch))


_agg_deg = _make_agg(True)
_agg = _make_agg(False)


def _layer_body(p_ref, degp_ref, x_ref, wl_ref, wr_ref, b_ref, o_ref):
    deg = jnp.sum(degp_ref[...], axis=0)
    inv = 1.0 / jnp.maximum(deg, 1.0)
    mean = (p_ref[0] + p_ref[1]) * inv[:, None]
    h = jnp.dot(mean, wl_ref[...]) + jnp.dot(x_ref[...], wr_ref[...]) + b_ref[...]
    o_ref[...] = jnp.maximum(h, 0.0)


def _layer(partials, deg_parts, x, wl, wr, b):
    return pl.pallas_call(
        _layer_body,
        grid=(NB,),
        in_specs=[
            pl.BlockSpec((NC, R, D), lambda i: (0, i, 0)),
            pl.BlockSpec((NC, R), lambda i: (0, i)),
            pl.BlockSpec((R, D), lambda i: (i, 0)),
            pl.BlockSpec((D, D), lambda i: (0, 0)),
            pl.BlockSpec((D, D), lambda i: (0, 0)),
            pl.BlockSpec((1, D), lambda i: (0, 0)),
        ],
        out_specs=pl.BlockSpec((R, D), lambda i: (i, 0)),
        out_shape=jax.ShapeDtypeStruct((N_PAD, D), jnp.float32),
    )(partials, deg_parts, x, wl, wr, b)


def _final_body(p_ref, degp_ref, h_ref, wl_ref, wr_ref, b_ref, batch_ref,
                w1_ref, b1_ref, w2_ref, b2_ref, o_ref, pool_scr, cnt_scr):
    i = pl.program_id(0)
    deg = jnp.sum(degp_ref[...], axis=0)
    inv = 1.0 / jnp.maximum(deg, 1.0)
    mean = (p_ref[0] + p_ref[1]) * inv[:, None]
    h3 = jnp.maximum(
        jnp.dot(mean, wl_ref[...]) + jnp.dot(h_ref[...], wr_ref[...])
        + b_ref[...], 0.0)
    bvec = batch_ref[0, 0, :]
    oh = (bvec[None, :] == lax.broadcasted_iota(jnp.int32, (G, R), 0)
          ).astype(jnp.float32)
    pool_upd = jnp.dot(oh, h3)
    cnt_upd = jnp.sum(oh, axis=1)[None, :]

    @pl.when(i == 0)
    def _():
        pool_scr[...] = pool_upd
        cnt_scr[...] = cnt_upd

    @pl.when(i > 0)
    def _():
        pool_scr[...] += pool_upd
        cnt_scr[...] += cnt_upd

    @pl.when(i == NB - 1)
    def _():
        pooled = pool_scr[...] / jnp.maximum(cnt_scr[0, :], 1.0)[:, None]
        z = jnp.maximum(jnp.dot(pooled, w1_ref[...]) + b1_ref[...], 0.0)
        logits = jnp.dot(z, w2_ref[...]) + b2_ref[...]
        m = jnp.max(logits, axis=-1, keepdims=True)
        s = jnp.log(jnp.sum(jnp.exp(logits - m), axis=-1, keepdims=True))
        o_ref[...] = logits - m - s


def _final(partials, deg_parts, h, wl, wr, b, batch3d, w1, b1, w2, b2):
    return pl.pallas_call(
        _final_body,
        grid=(NB,),
        in_specs=[
            pl.BlockSpec((NC, R, D), lambda i: (0, i, 0)),
            pl.BlockSpec((NC, R), lambda i: (0, i)),
            pl.BlockSpec((R, D), lambda i: (i, 0)),
            pl.BlockSpec((D, D), lambda i: (0, 0)),
            pl.BlockSpec((D, D), lambda i: (0, 0)),
            pl.BlockSpec((1, D), lambda i: (0, 0)),
            pl.BlockSpec((1, 1, R), lambda i: (i, 0, 0)),
            pl.BlockSpec((D, D), lambda i: (0, 0)),
            pl.BlockSpec((1, D), lambda i: (0, 0)),
            pl.BlockSpec((D, CLS), lambda i: (0, 0)),
            pl.BlockSpec((1, CLS), lambda i: (0, 0)),
        ],
        out_specs=pl.BlockSpec((G, CLS), lambda i: (0, 0)),
        out_shape=jax.ShapeDtypeStruct((G, CLS), jnp.float32),
        scratch_shapes=[
            pltpu.VMEM((G, D), jnp.float32),
            pltpu.VMEM((1, G), jnp.float32),
        ],
    )(partials, deg_parts, h, wl, wr, b, batch3d, w1, b1, w2, b2)


def kernel(x, edge_index, batch, Wl1, Wr1, b1, Wl2, Wr2, b2, Wl3, Wr3, b3,
           W_lin1, b_lin1, W_lin2, b_lin2):
    srcr = edge_index[0].reshape(NW, NGRP, BLK, CHUNK)
    dstr = edge_index[1].reshape(NW, NGRP, BLK, CHUNK)
    x_pad = jnp.pad(x, ((0, N_PAD - N), (0, 0)))
    batch3d = jnp.pad(batch, (0, N_PAD - N), constant_values=G
                      ).reshape(NB, 1, R)

    agg1, degp = _agg_deg(x_pad, srcr, dstr)
    h1 = _layer(agg1, degp, x_pad, Wl1, Wr1, b1.reshape(1, D))
    agg2 = _agg(h1, srcr, dstr)
    h2 = _layer(agg2, degp, h1, Wl2, Wr2, b2.reshape(1, D))
    agg3 = _agg(h2, srcr, dstr)
    return _final(agg3, degp, h2, Wl3, Wr3, b3.reshape(1, D), batch3d,
                  W_lin1, b_lin1.reshape(1, D), W_lin2, b_lin2.reshape(1, CLS))
